# baseline (speedup 1.0000x reference)
import functools

import jax
import jax.numpy as jnp
from jax import lax
from jax.experimental import pallas as pl
from jax.experimental.pallas import tpu as pltpu
from jax.experimental.pallas import tpu_sc as plsc

_N = 10000
_E = 320000
_NPAD = 10240
_NW = 32
_K = 125
_CH = _E // _NW // _K
_BLK = 2048
_ROWS_PER_TILE = _NPAD // 16

_MESH = plsc.VectorSubcoreMesh(core_axis_name="c", subcore_axis_name="s")
_SC_PARAMS = pltpu.CompilerParams(use_tc_tiling_on_sc=False)


@functools.partial(
    pl.kernel,
    mesh=_MESH,
    out_type=jax.ShapeDtypeStruct((2, _NPAD, 16), jnp.float32),
    scratch_types=[
        pltpu.VMEM((_CH, _K), jnp.int32),
        pltpu.VMEM((_K, 16), jnp.float32),
        pltpu.VMEM_SHARED((_NPAD, 16), jnp.float32),
        pltpu.SemaphoreType.DMA,
    ],
    compiler_params=_SC_PARAMS,
)
def _hist(dst_hbm, zeros_hbm, out_hbm, dstv, ones, acc, hsem):
    c = lax.axis_index("c")
    s = lax.axis_index("s")
    wid = c * 16 + s
    pltpu.sync_copy(dst_hbm.at[wid], dstv)

    def fill(i, carry):
        ones[i] = jnp.ones((16,), jnp.float32)
        return carry

    lax.fori_loop(0, _K, fill, 0)
    r0 = s * _ROWS_PER_TILE
    pltpu.sync_copy(zeros_hbm.at[pl.ds(r0, _ROWS_PER_TILE)],
                    acc.at[pl.ds(r0, _ROWS_PER_TILE)])
    plsc.subcore_barrier()

    def body(g, carry):
        for b in range(8):
            pltpu.async_copy(ones, acc.at[dstv.at[g * 8 + b]], hsem, add=True)
        for b in range(8):
            pltpu.make_async_copy(ones, acc.at[dstv.at[g * 8 + b]],
                                  hsem).wait()
        return carry

    lax.fori_loop(0, _CH // 8, body, 0)
    plsc.subcore_barrier()
    pltpu.sync_copy(acc.at[pl.ds(r0, _ROWS_PER_TILE)],
                    out_hbm.at[c, pl.ds(r0, _ROWS_PER_TILE)])


def _make_prop(d, k, ch, m, nbuf):
    ng = ch // m

    @functools.partial(
        pl.kernel,
        mesh=_MESH,
        out_type=jax.ShapeDtypeStruct((2, _NPAD, d), jnp.float32),
        scratch_types=(
            [pltpu.VMEM((ch, k), jnp.int32),
             pltpu.VMEM((ch, k), jnp.int32)]
            + [pltpu.VMEM((k, d), jnp.float32)] * nbuf
            + [pltpu.VMEM_SHARED((_NPAD, d), jnp.float32)]
            + [pltpu.SemaphoreType.DMA] * (2 * nbuf)
        ),
        compiler_params=_SC_PARAMS,
    )
    def prop(src_hbm, dst_hbm, hws_hbm, zeros_hbm, out_hbm, *sc):
        srcv, dstv = sc[0], sc[1]
        bufs = sc[2:2 + nbuf]
        acc = sc[2 + nbuf]
        gsems = sc[3 + nbuf:3 + 2 * nbuf]
        ssems = sc[3 + 2 * nbuf:3 + 3 * nbuf]
        c = lax.axis_index("c")
        s = lax.axis_index("s")
        wid = c * 16 + s
        pltpu.sync_copy(src_hbm.at[wid], srcv)
        pltpu.sync_copy(dst_hbm.at[wid], dstv)
        r0 = s * _ROWS_PER_TILE

        @pl.when(c == 0)
        def _():
            pltpu.sync_copy(hws_hbm.at[pl.ds(r0, _ROWS_PER_TILE)],
                            acc.at[pl.ds(r0, _ROWS_PER_TILE)])

        @pl.when(c != 0)
        def _():
            pltpu.sync_copy(zeros_hbm.at[pl.ds(r0, _ROWS_PER_TILE)],
                            acc.at[pl.ds(r0, _ROWS_PER_TILE)])

        plsc.subcore_barrier()

        for b in range(nbuf):
            pltpu.async_copy(hws_hbm.at[srcv.at[b]], bufs[b], gsems[b])

        def group(g, carry):
            cg0 = g * nbuf
            for b in range(nbuf):
                cg = cg0 + b
                pltpu.make_async_copy(hws_hbm.at[srcv.at[cg]],
                                      bufs[b], gsems[b]).wait()
                pltpu.async_copy(bufs[b], acc.at[dstv.at[cg]],
                                 ssems[b], add=True)
            for b in range(nbuf):
                cg = cg0 + b
                pltpu.make_async_copy(bufs[b], acc.at[dstv.at[cg]],
                                      ssems[b]).wait()

                @pl.when(cg + nbuf < ng)
                def _():
                    pltpu.async_copy(hws_hbm.at[srcv.at[cg + nbuf]],
                                     bufs[b], gsems[b])
            return carry

        lax.fori_loop(0, ng // nbuf, group, 0)
        plsc.subcore_barrier()
        pltpu.sync_copy(acc.at[pl.ds(r0, _ROWS_PER_TILE)],
                        out_hbm.at[c, pl.ds(r0, _ROWS_PER_TILE)])

    return prop


_K128, _CH128 = 50, 200
_K64, _CH64 = 125, 80
_prop128 = _make_prop(128, _K128, _CH128, 1, 4)
_prop64 = _make_prop(64, _K64, _CH64, 1, 4)


def _dinv_of(h_ref):
    deg = (h_ref[0] + h_ref[1]).sum(axis=-1) * (1.0 / 16.0) + 1.0
    return lax.rsqrt(deg)[:, None]


_HIST_SPEC = pl.BlockSpec((2, _BLK, 16), lambda i: (0, i, 0))


def _tc_first(h, W, hist):
    d_in, d_out = W.shape

    def body(h_ref, w_ref, hist_ref, o_ref):
        hw = jnp.dot(h_ref[...], w_ref[...], preferred_element_type=jnp.float32)
        o_ref[...] = hw * _dinv_of(hist_ref)

    return pl.pallas_call(
        body,
        grid=(_NPAD // _BLK,),
        in_specs=[
            pl.BlockSpec((_BLK, d_in), lambda i: (i, 0)),
            pl.BlockSpec((d_in, d_out), lambda i: (0, 0)),
            _HIST_SPEC,
        ],
        out_specs=pl.BlockSpec((_BLK, d_out), lambda i: (i, 0)),
        out_shape=jax.ShapeDtypeStruct((_NPAD, d_out), jnp.float32),
    )(h, W, hist)


def _tc_mid(p, b, W, hist):
    d_in, d_out = W.shape

    def body(p_ref, b_ref, w_ref, hist_ref, o_ref):
        dv = _dinv_of(hist_ref)
        h = dv * (p_ref[0] + p_ref[1]) + b_ref[...]
        h = jnp.maximum(h, 0.0)
        o_ref[...] = jnp.dot(h, w_ref[...],
                             preferred_element_type=jnp.float32) * dv

    return pl.pallas_call(
        body,
        grid=(_NPAD // _BLK,),
        in_specs=[
            pl.BlockSpec((2, _BLK, d_in), lambda i: (0, i, 0)),
            pl.BlockSpec((1, d_in), lambda i: (0, 0)),
            pl.BlockSpec((d_in, d_out), lambda i: (0, 0)),
            _HIST_SPEC,
        ],
        out_specs=pl.BlockSpec((_BLK, d_out), lambda i: (i, 0)),
        out_shape=jax.ShapeDtypeStruct((_NPAD, d_out), jnp.float32),
    )(p, b, W, hist)


def _tc_fuse(pg, bg, q, bl, hist, fW, fb, LC):
    def body(pg_ref, bg_ref, q_ref, bl_ref, hist_ref, fw_ref, fb_ref,
             lc_ref, o_ref):
        dv = _dinv_of(hist_ref)
        hgv = dv * (pg_ref[0] + pg_ref[1]) + bg_ref[...]
        hlv = dv * (q_ref[0] + q_ref[1]) + bl_ref[...]
        fw = fw_ref[...]
        fused = (jnp.dot(hgv, fw[:128], preferred_element_type=jnp.float32)
                 + jnp.dot(hlv, fw[128:], preferred_element_type=jnp.float32)
                 + fb_ref[...])
        out = jnp.dot(fused, lc_ref[...], preferred_element_type=jnp.float32)
        o_ref[...] = jax.nn.sigmoid(out)

    return pl.pallas_call(
        body,
        grid=(_NPAD // _BLK,),
        in_specs=[
            pl.BlockSpec((2, _BLK, 128), lambda i: (0, i, 0)),
            pl.BlockSpec((1, 128), lambda i: (0, 0)),
            pl.BlockSpec((2, _BLK, 64), lambda i: (0, i, 0)),
            pl.BlockSpec((1, 64), lambda i: (0, 0)),
            _HIST_SPEC,
            pl.BlockSpec((192, 64), lambda i: (0, 0)),
            pl.BlockSpec((1, 64), lambda i: (0, 0)),
            pl.BlockSpec((64, 64), lambda i: (0, 0)),
        ],
        out_specs=pl.BlockSpec((_BLK, 64), lambda i: (i, 0)),
        out_shape=jax.ShapeDtypeStruct((_NPAD, 64), jnp.float32),
    )(pg, bg, q, bl, hist, fW, fb, LC)


def kernel(x, y, edge_index, LC_matrix, gcn_W, gcn_b, label_W, label_b,
           fusion_W, fusion_b):
    s128 = edge_index[0].reshape(_NW, _CH128, _K128)
    d128 = edge_index[1].reshape(_NW, _CH128, _K128)
    s64 = edge_index[0].reshape(_NW, _CH64, _K64)
    d64 = edge_index[1].reshape(_NW, _CH64, _K64)
    dhist = edge_index[1].reshape(_NW, _CH, _K)
    xp = jnp.pad(x, ((0, _NPAD - _N), (0, 0)))
    yp = jnp.pad(y, ((0, _NPAD - _N), (0, 0)))
    z128 = jnp.zeros((_NPAD, 128), jnp.float32)
    z64 = jnp.zeros((_NPAD, 64), jnp.float32)
    z16 = jnp.zeros((_NPAD, 16), jnp.float32)

    hist = _hist(dhist, z16)

    hg = _tc_first(xp, gcn_W[0], hist)
    p = _prop128(s128, d128, hg, z128)
    hg = _tc_mid(p, gcn_b[0][None], gcn_W[1], hist)
    pg = _prop128(s128, d128, hg, z128)

    hl = _tc_first(yp, label_W[0], hist)
    q = _prop64(s64, d64, hl, z64)
    for j in range(1, 10):
        hl = _tc_mid(q, label_b[j - 1][None], label_W[j], hist)
        q = _prop64(s64, d64, hl, z64)

    out = _tc_fuse(pg, gcn_b[1][None],
                   q, label_b[9][None],
                   hist, fusion_W, fusion_b[None], LC_matrix)
    return out[:_N]


# speedup vs baseline: 1.0322x; 1.0322x over previous
import functools

import jax
import jax.numpy as jnp
from jax import lax
from jax.experimental import pallas as pl
from jax.experimental.pallas import tpu as pltpu
from jax.experimental.pallas import tpu_sc as plsc

_N = 10000
_E = 320000
_NPAD = 10240
_NW = 32
_K = 125
_CH = _E // _NW // _K
_BLK = 5120
_ROWS_PER_TILE = _NPAD // 16

_MESH = plsc.VectorSubcoreMesh(core_axis_name="c", subcore_axis_name="s")
_SC_PARAMS = pltpu.CompilerParams(use_tc_tiling_on_sc=False)


@functools.partial(
    pl.kernel,
    mesh=_MESH,
    out_type=jax.ShapeDtypeStruct((2, _NPAD, 16), jnp.float32),
    scratch_types=[
        pltpu.VMEM((_CH, _K), jnp.int32),
        pltpu.VMEM((_K, 16), jnp.float32),
        pltpu.VMEM_SHARED((_NPAD, 16), jnp.float32),
        pltpu.SemaphoreType.DMA,
    ],
    compiler_params=_SC_PARAMS,
)
def _hist(dst_hbm, zeros_hbm, out_hbm, dstv, ones, acc, hsem):
    c = lax.axis_index("c")
    s = lax.axis_index("s")
    wid = c * 16 + s
    pltpu.sync_copy(dst_hbm.at[wid], dstv)

    def fill(i, carry):
        ones[i] = jnp.ones((16,), jnp.float32)
        return carry

    lax.fori_loop(0, _K, fill, 0)
    r0 = s * _ROWS_PER_TILE
    pltpu.sync_copy(zeros_hbm.at[pl.ds(r0, _ROWS_PER_TILE)],
                    acc.at[pl.ds(r0, _ROWS_PER_TILE)])
    plsc.subcore_barrier()

    def body(g, carry):
        for b in range(8):
            pltpu.async_copy(ones, acc.at[dstv.at[g * 8 + b]], hsem, add=True)
        for b in range(8):
            pltpu.make_async_copy(ones, acc.at[dstv.at[g * 8 + b]],
                                  hsem).wait()
        return carry

    lax.fori_loop(0, _CH // 8, body, 0)
    plsc.subcore_barrier()
    pltpu.sync_copy(acc.at[pl.ds(r0, _ROWS_PER_TILE)],
                    out_hbm.at[c, pl.ds(r0, _ROWS_PER_TILE)])


def _make_prop(d, k, ch, m, nbuf):
    ng = ch // m

    @functools.partial(
        pl.kernel,
        mesh=_MESH,
        out_type=jax.ShapeDtypeStruct((2, _NPAD, d), jnp.float32),
        scratch_types=(
            [pltpu.VMEM((ch, k), jnp.int32),
             pltpu.VMEM((ch, k), jnp.int32)]
            + [pltpu.VMEM((k, d), jnp.float32)] * nbuf
            + [pltpu.VMEM_SHARED((_NPAD, d), jnp.float32)]
            + [pltpu.SemaphoreType.DMA] * (2 * nbuf)
        ),
        compiler_params=_SC_PARAMS,
    )
    def prop(src_hbm, dst_hbm, hws_hbm, zeros_hbm, out_hbm, *sc):
        srcv, dstv = sc[0], sc[1]
        bufs = sc[2:2 + nbuf]
        acc = sc[2 + nbuf]
        gsems = sc[3 + nbuf:3 + 2 * nbuf]
        ssems = sc[3 + 2 * nbuf:3 + 3 * nbuf]
        c = lax.axis_index("c")
        s = lax.axis_index("s")
        wid = c * 16 + s
        pltpu.sync_copy(src_hbm.at[wid], srcv)
        pltpu.sync_copy(dst_hbm.at[wid], dstv)
        r0 = s * _ROWS_PER_TILE

        @pl.when(c == 0)
        def _():
            pltpu.sync_copy(hws_hbm.at[pl.ds(r0, _ROWS_PER_TILE)],
                            acc.at[pl.ds(r0, _ROWS_PER_TILE)])

        @pl.when(c != 0)
        def _():
            pltpu.sync_copy(zeros_hbm.at[pl.ds(r0, _ROWS_PER_TILE)],
                            acc.at[pl.ds(r0, _ROWS_PER_TILE)])

        plsc.subcore_barrier()

        for b in range(nbuf):
            pltpu.async_copy(hws_hbm.at[srcv.at[b]], bufs[b], gsems[b])

        def group(g, carry):
            cg0 = g * nbuf
            for b in range(nbuf):
                cg = cg0 + b
                pltpu.make_async_copy(hws_hbm.at[srcv.at[cg]],
                                      bufs[b], gsems[b]).wait()
                pltpu.async_copy(bufs[b], acc.at[dstv.at[cg]],
                                 ssems[b], add=True)
            for b in range(nbuf):
                cg = cg0 + b
                pltpu.make_async_copy(bufs[b], acc.at[dstv.at[cg]],
                                      ssems[b]).wait()

                @pl.when(cg + nbuf < ng)
                def _():
                    pltpu.async_copy(hws_hbm.at[srcv.at[cg + nbuf]],
                                     bufs[b], gsems[b])
            return carry

        lax.fori_loop(0, ng // nbuf, group, 0)
        plsc.subcore_barrier()
        pltpu.sync_copy(acc.at[pl.ds(r0, _ROWS_PER_TILE)],
                        out_hbm.at[c, pl.ds(r0, _ROWS_PER_TILE)])

    return prop


_K128, _CH128 = 50, 200
_K64, _CH64 = 125, 80
_prop128 = _make_prop(128, _K128, _CH128, 1, 4)
_prop64 = _make_prop(64, _K64, _CH64, 1, 8)


def _dinv_of(h_ref):
    deg = (h_ref[0] + h_ref[1]).sum(axis=-1) * (1.0 / 16.0) + 1.0
    return lax.rsqrt(deg)[:, None]


_HIST_SPEC = pl.BlockSpec((2, _BLK, 16), lambda i: (0, i, 0))


def _tc_first(h, W, hist):
    d_in, d_out = W.shape

    def body(h_ref, w_ref, hist_ref, o_ref):
        hw = jnp.dot(h_ref[...], w_ref[...], preferred_element_type=jnp.float32)
        o_ref[...] = hw * _dinv_of(hist_ref)

    return pl.pallas_call(
        body,
        grid=(_NPAD // _BLK,),
        in_specs=[
            pl.BlockSpec((_BLK, d_in), lambda i: (i, 0)),
            pl.BlockSpec((d_in, d_out), lambda i: (0, 0)),
            _HIST_SPEC,
        ],
        out_specs=pl.BlockSpec((_BLK, d_out), lambda i: (i, 0)),
        out_shape=jax.ShapeDtypeStruct((_NPAD, d_out), jnp.float32),
    )(h, W, hist)


def _tc_mid(p, b, W, hist):
    d_in, d_out = W.shape

    def body(p_ref, b_ref, w_ref, hist_ref, o_ref):
        dv = _dinv_of(hist_ref)
        h = dv * (p_ref[0] + p_ref[1]) + b_ref[...]
        h = jnp.maximum(h, 0.0)
        o_ref[...] = jnp.dot(h, w_ref[...],
                             preferred_element_type=jnp.float32) * dv

    return pl.pallas_call(
        body,
        grid=(_NPAD // _BLK,),
        in_specs=[
            pl.BlockSpec((2, _BLK, d_in), lambda i: (0, i, 0)),
            pl.BlockSpec((1, d_in), lambda i: (0, 0)),
            pl.BlockSpec((d_in, d_out), lambda i: (0, 0)),
            _HIST_SPEC,
        ],
        out_specs=pl.BlockSpec((_BLK, d_out), lambda i: (i, 0)),
        out_shape=jax.ShapeDtypeStruct((_NPAD, d_out), jnp.float32),
    )(p, b, W, hist)


def _tc_fuse(pg, bg, q, bl, hist, fW, fb, LC):
    def body(pg_ref, bg_ref, q_ref, bl_ref, hist_ref, fw_ref, fb_ref,
             lc_ref, o_ref):
        dv = _dinv_of(hist_ref)
        hgv = dv * (pg_ref[0] + pg_ref[1]) + bg_ref[...]
        hlv = dv * (q_ref[0] + q_ref[1]) + bl_ref[...]
        fw = fw_ref[...]
        fused = (jnp.dot(hgv, fw[:128], preferred_element_type=jnp.float32)
                 + jnp.dot(hlv, fw[128:], preferred_element_type=jnp.float32)
                 + fb_ref[...])
        out = jnp.dot(fused, lc_ref[...], preferred_element_type=jnp.float32)
        o_ref[...] = jax.nn.sigmoid(out)

    return pl.pallas_call(
        body,
        grid=(_NPAD // _BLK,),
        in_specs=[
            pl.BlockSpec((2, _BLK, 128), lambda i: (0, i, 0)),
            pl.BlockSpec((1, 128), lambda i: (0, 0)),
            pl.BlockSpec((2, _BLK, 64), lambda i: (0, i, 0)),
            pl.BlockSpec((1, 64), lambda i: (0, 0)),
            _HIST_SPEC,
            pl.BlockSpec((192, 64), lambda i: (0, 0)),
            pl.BlockSpec((1, 64), lambda i: (0, 0)),
            pl.BlockSpec((64, 64), lambda i: (0, 0)),
        ],
        out_specs=pl.BlockSpec((_BLK, 64), lambda i: (i, 0)),
        out_shape=jax.ShapeDtypeStruct((_NPAD, 64), jnp.float32),
    )(pg, bg, q, bl, hist, fW, fb, LC)


def kernel(x, y, edge_index, LC_matrix, gcn_W, gcn_b, label_W, label_b,
           fusion_W, fusion_b):
    s128 = edge_index[0].reshape(_NW, _CH128, _K128)
    d128 = edge_index[1].reshape(_NW, _CH128, _K128)
    s64 = edge_index[0].reshape(_NW, _CH64, _K64)
    d64 = edge_index[1].reshape(_NW, _CH64, _K64)
    dhist = edge_index[1].reshape(_NW, _CH, _K)
    xp = jnp.pad(x, ((0, _NPAD - _N), (0, 0)))
    yp = jnp.pad(y, ((0, _NPAD - _N), (0, 0)))
    z128 = jnp.zeros((_NPAD, 128), jnp.float32)
    z64 = jnp.zeros((_NPAD, 64), jnp.float32)
    z16 = jnp.zeros((_NPAD, 16), jnp.float32)

    hist = _hist(dhist, z16)

    hg = _tc_first(xp, gcn_W[0], hist)
    p = _prop128(s128, d128, hg, z128)
    hg = _tc_mid(p, gcn_b[0][None], gcn_W[1], hist)
    pg = _prop128(s128, d128, hg, z128)

    hl = _tc_first(yp, label_W[0], hist)
    q = _prop64(s64, d64, hl, z64)
    for j in range(1, 10):
        hl = _tc_mid(q, label_b[j - 1][None], label_W[j], hist)
        q = _prop64(s64, d64, hl, z64)

    out = _tc_fuse(pg, gcn_b[1][None],
                   q, label_b[9][None],
                   hist, fusion_W, fusion_b[None], LC_matrix)
    return out[:_N]


# speedup vs baseline: 1.0415x; 1.0090x over previous
import functools

import jax
import jax.numpy as jnp
from jax import lax
from jax.experimental import pallas as pl
from jax.experimental.pallas import tpu as pltpu
from jax.experimental.pallas import tpu_sc as plsc

_N = 10000
_E = 320000
_NPAD = 10000
_NW = 32
_K = 125
_CH = _E // _NW // _K
_BLK = 2000
_ROWS_PER_TILE = _NPAD // 16

_MESH = plsc.VectorSubcoreMesh(core_axis_name="c", subcore_axis_name="s")
_SC_PARAMS = pltpu.CompilerParams(use_tc_tiling_on_sc=False)


@functools.partial(
    pl.kernel,
    mesh=_MESH,
    out_type=jax.ShapeDtypeStruct((2, _NPAD, 16), jnp.float32),
    scratch_types=[
        pltpu.VMEM((_CH, _K), jnp.int32),
        pltpu.VMEM((_K, 16), jnp.float32),
        pltpu.VMEM_SHARED((_NPAD, 16), jnp.float32),
        pltpu.SemaphoreType.DMA,
    ],
    compiler_params=_SC_PARAMS,
)
def _hist(dst_hbm, zeros_hbm, out_hbm, dstv, ones, acc, hsem):
    c = lax.axis_index("c")
    s = lax.axis_index("s")
    wid = c * 16 + s
    pltpu.sync_copy(dst_hbm.at[wid], dstv)

    def fill(i, carry):
        ones[i] = jnp.ones((16,), jnp.float32)
        return carry

    lax.fori_loop(0, _K, fill, 0)
    r0 = s * _ROWS_PER_TILE
    pltpu.sync_copy(zeros_hbm.at[pl.ds(r0, _ROWS_PER_TILE)],
                    acc.at[pl.ds(r0, _ROWS_PER_TILE)])
    plsc.subcore_barrier()

    def body(g, carry):
        for b in range(8):
            pltpu.async_copy(ones, acc.at[dstv.at[g * 8 + b]], hsem, add=True)
        for b in range(8):
            pltpu.make_async_copy(ones, acc.at[dstv.at[g * 8 + b]],
                                  hsem).wait()
        return carry

    lax.fori_loop(0, _CH // 8, body, 0)
    plsc.subcore_barrier()
    pltpu.sync_copy(acc.at[pl.ds(r0, _ROWS_PER_TILE)],
                    out_hbm.at[c, pl.ds(r0, _ROWS_PER_TILE)])


def _make_prop(d, k, ch, m, nbuf):
    ng = ch // m

    @functools.partial(
        pl.kernel,
        mesh=_MESH,
        out_type=jax.ShapeDtypeStruct((2, _NPAD, d), jnp.float32),
        scratch_types=(
            [pltpu.VMEM((ch, k), jnp.int32),
             pltpu.VMEM((ch, k), jnp.int32)]
            + [pltpu.VMEM((k, d), jnp.float32)] * nbuf
            + [pltpu.VMEM_SHARED((_NPAD, d), jnp.float32)]
            + [pltpu.SemaphoreType.DMA] * (2 * nbuf)
        ),
        compiler_params=_SC_PARAMS,
    )
    def prop(src_hbm, dst_hbm, hws_hbm, zeros_hbm, out_hbm, *sc):
        srcv, dstv = sc[0], sc[1]
        bufs = sc[2:2 + nbuf]
        acc = sc[2 + nbuf]
        gsems = sc[3 + nbuf:3 + 2 * nbuf]
        ssems = sc[3 + 2 * nbuf:3 + 3 * nbuf]
        c = lax.axis_index("c")
        s = lax.axis_index("s")
        wid = c * 16 + s
        pltpu.sync_copy(src_hbm.at[wid], srcv)
        pltpu.sync_copy(dst_hbm.at[wid], dstv)
        r0 = s * _ROWS_PER_TILE

        @pl.when(c == 0)
        def _():
            pltpu.sync_copy(hws_hbm.at[pl.ds(r0, _ROWS_PER_TILE)],
                            acc.at[pl.ds(r0, _ROWS_PER_TILE)])

        @pl.when(c != 0)
        def _():
            pltpu.sync_copy(zeros_hbm.at[pl.ds(r0, _ROWS_PER_TILE)],
                            acc.at[pl.ds(r0, _ROWS_PER_TILE)])

        plsc.subcore_barrier()

        for b in range(nbuf):
            pltpu.async_copy(hws_hbm.at[srcv.at[b]], bufs[b], gsems[b])

        def group(g, carry):
            cg0 = g * nbuf
            for b in range(nbuf):
                cg = cg0 + b
                pltpu.make_async_copy(hws_hbm.at[srcv.at[cg]],
                                      bufs[b], gsems[b]).wait()
                pltpu.async_copy(bufs[b], acc.at[dstv.at[cg]],
                                 ssems[b], add=True)
            for b in range(nbuf):
                cg = cg0 + b
                pltpu.make_async_copy(bufs[b], acc.at[dstv.at[cg]],
                                      ssems[b]).wait()

                @pl.when(cg + nbuf < ng)
                def _():
                    pltpu.async_copy(hws_hbm.at[srcv.at[cg + nbuf]],
                                     bufs[b], gsems[b])
            return carry

        lax.fori_loop(0, ng // nbuf, group, 0)
        plsc.subcore_barrier()
        pltpu.sync_copy(acc.at[pl.ds(r0, _ROWS_PER_TILE)],
                        out_hbm.at[c, pl.ds(r0, _ROWS_PER_TILE)])

    return prop


_K128, _CH128 = 50, 200
_K64, _CH64 = 125, 80
_prop128 = _make_prop(128, _K128, _CH128, 1, 4)
_prop64 = _make_prop(64, _K64, _CH64, 1, 8)


def _dinv_of(h_ref):
    deg = (h_ref[0] + h_ref[1]).sum(axis=-1) * (1.0 / 16.0) + 1.0
    return lax.rsqrt(deg)[:, None]


_HIST_SPEC = pl.BlockSpec((2, _BLK, 16), lambda i: (0, i, 0))


def _tc_first(h, W, hist):
    d_in, d_out = W.shape

    def body(h_ref, w_ref, hist_ref, o_ref):
        hw = jnp.dot(h_ref[...], w_ref[...], preferred_element_type=jnp.float32)
        o_ref[...] = hw * _dinv_of(hist_ref)

    return pl.pallas_call(
        body,
        grid=(_NPAD // _BLK,),
        in_specs=[
            pl.BlockSpec((_BLK, d_in), lambda i: (i, 0)),
            pl.BlockSpec((d_in, d_out), lambda i: (0, 0)),
            _HIST_SPEC,
        ],
        out_specs=pl.BlockSpec((_BLK, d_out), lambda i: (i, 0)),
        out_shape=jax.ShapeDtypeStruct((_NPAD, d_out), jnp.float32),
    )(h, W, hist)


def _tc_mid(p, b, W, hist):
    d_in, d_out = W.shape

    def body(p_ref, b_ref, w_ref, hist_ref, o_ref):
        dv = _dinv_of(hist_ref)
        h = dv * (p_ref[0] + p_ref[1]) + b_ref[...]
        h = jnp.maximum(h, 0.0)
        o_ref[...] = jnp.dot(h, w_ref[...],
                             preferred_element_type=jnp.float32) * dv

    return pl.pallas_call(
        body,
        grid=(_NPAD // _BLK,),
        in_specs=[
            pl.BlockSpec((2, _BLK, d_in), lambda i: (0, i, 0)),
            pl.BlockSpec((1, d_in), lambda i: (0, 0)),
            pl.BlockSpec((d_in, d_out), lambda i: (0, 0)),
            _HIST_SPEC,
        ],
        out_specs=pl.BlockSpec((_BLK, d_out), lambda i: (i, 0)),
        out_shape=jax.ShapeDtypeStruct((_NPAD, d_out), jnp.float32),
    )(p, b, W, hist)


def _tc_fuse(pg, bg, q, bl, hist, fW, fb, LC):
    def body(pg_ref, bg_ref, q_ref, bl_ref, hist_ref, fw_ref, fb_ref,
             lc_ref, o_ref):
        dv = _dinv_of(hist_ref)
        hgv = dv * (pg_ref[0] + pg_ref[1]) + bg_ref[...]
        hlv = dv * (q_ref[0] + q_ref[1]) + bl_ref[...]
        fw = fw_ref[...]
        fused = (jnp.dot(hgv, fw[:128], preferred_element_type=jnp.float32)
                 + jnp.dot(hlv, fw[128:], preferred_element_type=jnp.float32)
                 + fb_ref[...])
        out = jnp.dot(fused, lc_ref[...], preferred_element_type=jnp.float32)
        o_ref[...] = jax.nn.sigmoid(out)

    return pl.pallas_call(
        body,
        grid=(_NPAD // _BLK,),
        in_specs=[
            pl.BlockSpec((2, _BLK, 128), lambda i: (0, i, 0)),
            pl.BlockSpec((1, 128), lambda i: (0, 0)),
            pl.BlockSpec((2, _BLK, 64), lambda i: (0, i, 0)),
            pl.BlockSpec((1, 64), lambda i: (0, 0)),
            _HIST_SPEC,
            pl.BlockSpec((192, 64), lambda i: (0, 0)),
            pl.BlockSpec((1, 64), lambda i: (0, 0)),
            pl.BlockSpec((64, 64), lambda i: (0, 0)),
        ],
        out_specs=pl.BlockSpec((_BLK, 64), lambda i: (i, 0)),
        out_shape=jax.ShapeDtypeStruct((_NPAD, 64), jnp.float32),
    )(pg, bg, q, bl, hist, fW, fb, LC)


def kernel(x, y, edge_index, LC_matrix, gcn_W, gcn_b, label_W, label_b,
           fusion_W, fusion_b):
    s128 = edge_index[0].reshape(_NW, _CH128, _K128)
    d128 = edge_index[1].reshape(_NW, _CH128, _K128)
    s64 = edge_index[0].reshape(_NW, _CH64, _K64)
    d64 = edge_index[1].reshape(_NW, _CH64, _K64)
    dhist = edge_index[1].reshape(_NW, _CH, _K)
    xp = x
    yp = y
    z128 = jnp.zeros((_NPAD, 128), jnp.float32)
    z64 = jnp.zeros((_NPAD, 64), jnp.float32)
    z16 = jnp.zeros((_NPAD, 16), jnp.float32)

    hist = _hist(dhist, z16)

    hg = _tc_first(xp, gcn_W[0], hist)
    p = _prop128(s128, d128, hg, z128)
    hg = _tc_mid(p, gcn_b[0][None], gcn_W[1], hist)
    pg = _prop128(s128, d128, hg, z128)

    hl = _tc_first(yp, label_W[0], hist)
    q = _prop64(s64, d64, hl, z64)
    for j in range(1, 10):
        hl = _tc_mid(q, label_b[j - 1][None], label_W[j], hist)
        q = _prop64(s64, d64, hl, z64)

    out = _tc_fuse(pg, gcn_b[1][None],
                   q, label_b[9][None],
                   hist, fusion_W, fusion_b[None], LC_matrix)
    return out
